# baseline (speedup 1.0000x reference)
import functools

import jax
import jax.numpy as jnp
from jax.experimental import pallas as pl
from jax.experimental.pallas import tpu as pltpu

_TI = 4096
_TC = 512


def _chamfer_body(x_ref, yt_ref, out_ref, colmin_ref, *,
                  ni, m, inv_xn, inv_ym):
    b = pl.program_id(0)
    i = pl.program_id(1)

    xs = x_ref[0]
    ys = yt_ref[0]

    x2 = jnp.sum(xs * xs, axis=1, keepdims=True)
    y2 = jnp.sum(ys * ys, axis=0, keepdims=True)
    x2h = x2.astype(jnp.bfloat16)
    x2l = (x2 - x2h.astype(jnp.float32)).astype(jnp.bfloat16)
    y2h = y2.astype(jnp.bfloat16)
    y2l = (y2 - y2h.astype(jnp.float32)).astype(jnp.bfloat16)
    ones_x = jnp.ones(x2h.shape, jnp.bfloat16)
    a = jnp.concatenate(
        [(xs * -2.0).astype(jnp.bfloat16), x2h, x2l, ones_x, ones_x],
        axis=1)
    ones_y = jnp.ones(y2h.shape, jnp.bfloat16)
    bmat = jnp.concatenate(
        [ys.astype(jnp.bfloat16), ones_y, ones_y, y2h, y2l],
        axis=0)

    rowmin = None
    cols = []
    for c in range(m // _TC):
        dc = jax.lax.dot_general(
            a, bmat[:, c * _TC:(c + 1) * _TC],
            dimension_numbers=(((1,), (0,)), ((), ())),
            preferred_element_type=jnp.float32)
        rc = jnp.min(dc, axis=1, keepdims=True)
        rowmin = rc if rowmin is None else jnp.minimum(rowmin, rc)
        cols.append(jnp.min(dc, axis=0, keepdims=True))
    col = jnp.concatenate(cols, axis=1)

    rowmin = jnp.maximum(rowmin, 0.0)

    @pl.when(jnp.logical_and(b == 0, i == 0))
    def _():
        out_ref[0, 0] = 0.0

    out_ref[0, 0] += jnp.sum(rowmin) * inv_xn

    @pl.when(i == 0)
    def _():
        colmin_ref[...] = col

    @pl.when(i > 0)
    def _():
        colmin_ref[...] = jnp.minimum(colmin_ref[...], col)

    @pl.when(i == ni - 1)
    def _():
        out_ref[0, 0] += jnp.sum(jnp.maximum(colmin_ref[...], 0.0)) * inv_ym


def kernel(x, y):
    B, N, _ = x.shape
    M = y.shape[1]
    ni = N // _TI
    yt = jnp.transpose(y, (0, 2, 1))

    out = pl.pallas_call(
        functools.partial(_chamfer_body, ni=ni, m=M,
                          inv_xn=1.0 / (N * B), inv_ym=1.0 / (M * B)),
        grid=(B, ni),
        in_specs=[
            pl.BlockSpec((1, _TI, 3), lambda b, i: (b, i, 0)),
            pl.BlockSpec((1, 3, M), lambda b, i: (b, 0, 0)),
        ],
        out_specs=pl.BlockSpec((1, 1), lambda b, i: (0, 0),
                               memory_space=pltpu.SMEM),
        out_shape=jax.ShapeDtypeStruct((1, 1), jnp.float32),
        scratch_shapes=[
            pltpu.VMEM((1, M), jnp.float32),
        ],
    )(x, yt)
    return out[0, 0]


# speedup vs baseline: 1.0384x; 1.0384x over previous
import functools

import jax
import jax.numpy as jnp
from jax.experimental import pallas as pl
from jax.experimental.pallas import tpu as pltpu

_TI = 4096
_TC = 2048


def _chamfer_body(x_ref, yt_ref, out_ref, colmin_ref, *,
                  ni, m, inv_xn, inv_ym):
    b = pl.program_id(0)
    i = pl.program_id(1)

    xs = x_ref[0]
    ys = yt_ref[0]

    x2 = jnp.sum(xs * xs, axis=1, keepdims=True)
    y2 = jnp.sum(ys * ys, axis=0, keepdims=True)
    x2h = x2.astype(jnp.bfloat16)
    x2l = (x2 - x2h.astype(jnp.float32)).astype(jnp.bfloat16)
    y2h = y2.astype(jnp.bfloat16)
    y2l = (y2 - y2h.astype(jnp.float32)).astype(jnp.bfloat16)
    ones_x = jnp.ones(x2h.shape, jnp.bfloat16)
    a = jnp.concatenate(
        [(xs * -2.0).astype(jnp.bfloat16), x2h, x2l, ones_x, ones_x],
        axis=1)
    ones_y = jnp.ones(y2h.shape, jnp.bfloat16)
    bmat = jnp.concatenate(
        [ys.astype(jnp.bfloat16), ones_y, ones_y, y2h, y2l],
        axis=0)

    rowmin = None
    cols = []
    for c in range(m // _TC):
        dc = jax.lax.dot_general(
            a, bmat[:, c * _TC:(c + 1) * _TC],
            dimension_numbers=(((1,), (0,)), ((), ())),
            preferred_element_type=jnp.float32)
        rc = jnp.min(dc, axis=1, keepdims=True)
        rowmin = rc if rowmin is None else jnp.minimum(rowmin, rc)
        cols.append(jnp.min(dc, axis=0, keepdims=True))
    col = jnp.concatenate(cols, axis=1)

    rowmin = jnp.maximum(rowmin, 0.0)

    @pl.when(jnp.logical_and(b == 0, i == 0))
    def _():
        out_ref[0, 0] = 0.0

    out_ref[0, 0] += jnp.sum(rowmin) * inv_xn

    @pl.when(i == 0)
    def _():
        colmin_ref[...] = col

    @pl.when(i > 0)
    def _():
        colmin_ref[...] = jnp.minimum(colmin_ref[...], col)

    @pl.when(i == ni - 1)
    def _():
        out_ref[0, 0] += jnp.sum(jnp.maximum(colmin_ref[...], 0.0)) * inv_ym


def kernel(x, y):
    B, N, _ = x.shape
    M = y.shape[1]
    ni = N // _TI
    yt = jnp.transpose(y, (0, 2, 1))

    out = pl.pallas_call(
        functools.partial(_chamfer_body, ni=ni, m=M,
                          inv_xn=1.0 / (N * B), inv_ym=1.0 / (M * B)),
        grid=(B, ni),
        in_specs=[
            pl.BlockSpec((1, _TI, 3), lambda b, i: (b, i, 0)),
            pl.BlockSpec((1, 3, M), lambda b, i: (b, 0, 0)),
        ],
        out_specs=pl.BlockSpec((1, 1), lambda b, i: (0, 0),
                               memory_space=pltpu.SMEM),
        out_shape=jax.ShapeDtypeStruct((1, 1), jnp.float32),
        scratch_shapes=[
            pltpu.VMEM((1, M), jnp.float32),
        ],
    )(x, yt)
    return out[0, 0]
